# baseline (speedup 1.0000x reference)
import functools

import jax
import jax.numpy as jnp
from jax import lax
from jax.experimental import pallas as pl
from jax.experimental.pallas import tpu as pltpu
from jax.experimental.pallas import tpu_sc as plsc

_N = 10000
_NPAD = 10240
_E = 320000
_D = 128
_CHUNK = 128
_NCHUNKS = _E // _CHUNK
_NW = 32
_RPT = _NPAD // 16

_NB = 2
_CPT = 80
_NCHUNKS_P = _CPT * _NW
_EPAD = _NCHUNKS_P * _CHUNK - _E

_mesh = plsc.VectorSubcoreMesh(core_axis_name="c", subcore_axis_name="s")


@functools.partial(
    pl.kernel,
    out_type=jax.ShapeDtypeStruct((2, _NPAD, _D), jnp.float32),
    mesh=_mesh,
    scratch_types=[
        pltpu.VMEM_SHARED((_NPAD, _D), jnp.float32),
        pltpu.VMEM((_NB, _CHUNK, _D), jnp.float32),
        pltpu.VMEM((2 * _NB, _CHUNK), jnp.int32),
        pltpu.VMEM((2 * _NB, _CHUNK), jnp.int32),
        pltpu.SemaphoreType.DMA((_NB,)),
        pltpu.SemaphoreType.DMA((2 * _NB,)),
        pltpu.SemaphoreType.DMA((2 * _NB,)),
        pltpu.SemaphoreType.DMA,
    ],
)
def _agg_kernel(hs_hbm, src2d_hbm, dst2d_hbm, zrows_hbm, out_hbm,
                acc, rows, sidx, didx, gsem, isem, dsem, ssem):
    cid = lax.axis_index("c")
    sid = lax.axis_index("s")
    base = sid * _RPT
    pltpu.sync_copy(zrows_hbm, acc.at[pl.ds(base, _RPT)])
    wid = sid * 2 + cid
    c0 = wid * _CPT
    plsc.subcore_barrier()

    def fire_idx(j, si):
        pltpu.async_copy(src2d_hbm.at[c0 + j], sidx.at[si], isem.at[si])
        pltpu.async_copy(dst2d_hbm.at[c0 + j], didx.at[si], dsem.at[si])

    def wait_idx(j, si):
        pltpu.make_async_copy(src2d_hbm.at[c0 + j], sidx.at[si],
                              isem.at[si]).wait()
        pltpu.make_async_copy(dst2d_hbm.at[c0 + j], didx.at[si],
                              dsem.at[si]).wait()

    def fire_gather(si, rs):
        pltpu.async_copy(hs_hbm.at[sidx.at[si]], rows.at[rs], gsem.at[rs])

    def wait_gather(si, rs):
        pltpu.make_async_copy(hs_hbm.at[sidx.at[si]], rows.at[rs],
                              gsem.at[rs]).wait()

    def scatter(si, rs):
        pltpu.async_copy(rows.at[rs], acc.at[didx.at[si]], ssem,
                         add=True).wait()

    def body(j, carry):
        off = (wid + j * _NW) * _CHUNK
        pltpu.sync_copy(src2d_hbm.at[pl.ds(off, _CHUNK)], sidx.at[0])
        pltpu.sync_copy(dst2d_hbm.at[pl.ds(off, _CHUNK)], didx.at[0])
        pltpu.async_copy(hs_hbm.at[sidx.at[0]], rows.at[0], gsem.at[0]).wait()
        pltpu.async_copy(rows.at[0], acc.at[didx.at[0]], ssem, add=True).wait()
        return carry

    lax.fori_loop(0, _CPT, body, 0)
    plsc.subcore_barrier()
    pltpu.sync_copy(acc.at[pl.ds(base, _RPT)], out_hbm.at[cid, pl.ds(base, _RPT)])


_R = 1000


def _tc_pre_body(deg_ref, x_ref, w_ref, hs_ref, dis_ref):
    deg = deg_ref[0, :, :1] + deg_ref[1, :, :1] + 1.0
    dis = lax.rsqrt(deg)
    dis_ref[...] = jnp.broadcast_to(dis, (_R, 16))
    xw = jnp.dot(x_ref[...], w_ref[...], preferred_element_type=jnp.float32)
    hs_ref[...] = xw * dis


def _tc_pre(degpair, x, W0):
    return pl.pallas_call(
        _tc_pre_body,
        grid=(_N // _R,),
        in_specs=[
            pl.BlockSpec((2, _R, _D), lambda i: (0, i, 0)),
            pl.BlockSpec((_R, _D), lambda i: (i, 0)),
            pl.BlockSpec((_D, _D), lambda i: (0, 0)),
        ],
        out_specs=[
            pl.BlockSpec((_R, _D), lambda i: (i, 0)),
            pl.BlockSpec((_R, 16), lambda i: (i, 0)),
        ],
        out_shape=[
            jax.ShapeDtypeStruct((_N, _D), jnp.float32),
            jax.ShapeDtypeStruct((_N, 16), jnp.float32),
        ],
    )(degpair, x, W0)


def _tc_mid_body(p_ref, hs_ref, dis_ref, b_ref, w_ref, o_ref):
    d = dis_ref[:, :1]
    agg = (p_ref[0] + p_ref[1] + hs_ref[...]) * d
    h = jnp.maximum(agg + b_ref[...], 0.0)
    o_ref[...] = jnp.dot(h, w_ref[...], preferred_element_type=jnp.float32) * d


def _tc_mid(p, hs, dis, b, W):
    return pl.pallas_call(
        _tc_mid_body,
        grid=(_N // _R,),
        in_specs=[
            pl.BlockSpec((2, _R, _D), lambda i: (0, i, 0)),
            pl.BlockSpec((_R, _D), lambda i: (i, 0)),
            pl.BlockSpec((_R, 16), lambda i: (i, 0)),
            pl.BlockSpec((1, _D), lambda i: (0, 0)),
            pl.BlockSpec((_D, _D), lambda i: (0, 0)),
        ],
        out_specs=pl.BlockSpec((_R, _D), lambda i: (i, 0)),
        out_shape=jax.ShapeDtypeStruct((_N, _D), jnp.float32),
    )(p, hs, dis, b.reshape(1, _D), W)


def _tc_fin_body(p_ref, hs_ref, dis_ref, b_ref, o_ref):
    d = dis_ref[:, :1]
    z = (p_ref[0] + p_ref[1] + hs_ref[...]) * d + b_ref[...]
    m = jnp.max(z, axis=1, keepdims=True)
    zs = z - m
    o_ref[...] = zs - jnp.log(jnp.sum(jnp.exp(zs), axis=1, keepdims=True))


def _tc_fin(p, hs, dis, b):
    return pl.pallas_call(
        _tc_fin_body,
        grid=(_N // _R,),
        in_specs=[
            pl.BlockSpec((2, _R, _D), lambda i: (0, i, 0)),
            pl.BlockSpec((_R, _D), lambda i: (i, 0)),
            pl.BlockSpec((_R, 16), lambda i: (i, 0)),
            pl.BlockSpec((1, _D), lambda i: (0, 0)),
        ],
        out_specs=pl.BlockSpec((_R, _D), lambda i: (i, 0)),
        out_shape=jax.ShapeDtypeStruct((_N, _D), jnp.float32),
    )(p, hs, dis, b.reshape(1, _D))


def kernel(x, edge_index, W0, b0, W1, b1, W2, b2):
    src = jnp.concatenate([edge_index[0], jnp.zeros((_EPAD,), jnp.int32)])
    dst = jnp.concatenate([edge_index[1], jnp.full((_EPAD,), _N, jnp.int32)])
    zrows = jnp.zeros((_RPT, _D), jnp.float32)
    ones_tbl = jnp.ones((_N, _D), jnp.float32)

    degpair = _agg_kernel(ones_tbl, src, dst, zrows)
    hs0, dis = _tc_pre(degpair, x, W0)
    p0 = _agg_kernel(hs0, src, dst, zrows)
    hs1 = _tc_mid(p0, hs0, dis, b0, W1)
    p1 = _agg_kernel(hs1, src, dst, zrows)
    hs2 = _tc_mid(p1, hs1, dis, b1, W2)
    p2 = _agg_kernel(hs2, src, dst, zrows)
    return _tc_fin(p2, hs2, dis, b2)


# speedup vs baseline: 2.4432x; 2.4432x over previous
import functools

import jax
import jax.numpy as jnp
from jax import lax
from jax.experimental import pallas as pl
from jax.experimental.pallas import tpu as pltpu
from jax.experimental.pallas import tpu_sc as plsc

_N = 10000
_NPAD = 10240
_E = 320000
_D = 128
_CHUNK = 128
_NCHUNKS = _E // _CHUNK
_NW = 32
_RPT = _NPAD // 16

_mesh = plsc.VectorSubcoreMesh(core_axis_name="c", subcore_axis_name="s")


@functools.partial(
    pl.kernel,
    out_type=jax.ShapeDtypeStruct((2, _NPAD, 16), jnp.float32),
    mesh=_mesh,
    compiler_params=pltpu.CompilerParams(use_tc_tiling_on_sc=False),
    scratch_types=[
        pltpu.VMEM_SHARED((_NPAD, 16), jnp.float32),
        pltpu.VMEM((_CHUNK, 16), jnp.float32),
        pltpu.VMEM((1, _CHUNK), jnp.int32),
        pltpu.SemaphoreType.DMA,
    ],
)
def _deg_kernel(dst_hbm, zeros16_hbm, ones16_hbm, out_hbm, acc, ones_v, didx, sem):
    cid = lax.axis_index("c")
    sid = lax.axis_index("s")
    base = sid * _RPT
    pltpu.sync_copy(zeros16_hbm, acc.at[pl.ds(base, _RPT)])
    pltpu.sync_copy(ones16_hbm, ones_v)
    plsc.subcore_barrier()
    wid = sid * 2 + cid

    def body(j, carry):
        off = (wid + j * _NW) * _CHUNK
        pltpu.sync_copy(dst_hbm.at[pl.ds(off, _CHUNK)], didx.at[0])
        pltpu.async_copy(ones_v, acc.at[didx.at[0]], sem, add=True).wait()
        return carry

    lax.fori_loop(0, (_NCHUNKS - wid + _NW - 1) // _NW, body, 0)
    plsc.subcore_barrier()
    pltpu.sync_copy(acc.at[pl.ds(base, _RPT)], out_hbm.at[cid, pl.ds(base, _RPT)])


@functools.partial(
    pl.kernel,
    out_type=jax.ShapeDtypeStruct((2, _NPAD, _D), jnp.float32),
    mesh=_mesh,
    scratch_types=[
        pltpu.VMEM_SHARED((_NPAD, _D), jnp.float32),
        pltpu.VMEM((_CHUNK, _D), jnp.float32),
        pltpu.VMEM((1, _CHUNK), jnp.int32),
        pltpu.VMEM((1, _CHUNK), jnp.int32),
        pltpu.SemaphoreType.DMA,
        pltpu.SemaphoreType.DMA,
    ],
)
def _agg_kernel(hs_hbm, src_hbm, dst_hbm, zrows_hbm, out_hbm,
                acc, rows, sidx, didx, gsem, ssem):
    cid = lax.axis_index("c")
    sid = lax.axis_index("s")
    base = sid * _RPT
    pltpu.sync_copy(zrows_hbm, acc.at[pl.ds(base, _RPT)])
    plsc.subcore_barrier()
    wid = sid * 2 + cid

    def body(j, carry):
        off = (wid + j * _NW) * _CHUNK
        pltpu.sync_copy(src_hbm.at[pl.ds(off, _CHUNK)], sidx.at[0])
        pltpu.sync_copy(dst_hbm.at[pl.ds(off, _CHUNK)], didx.at[0])
        pltpu.async_copy(hs_hbm.at[sidx.at[0]], rows, gsem).wait()
        pltpu.async_copy(rows, acc.at[didx.at[0]], ssem, add=True).wait()
        return carry

    lax.fori_loop(0, (_NCHUNKS - wid + _NW - 1) // _NW, body, 0)
    plsc.subcore_barrier()
    pltpu.sync_copy(acc.at[pl.ds(base, _RPT)], out_hbm.at[cid, pl.ds(base, _RPT)])


_R = 1000


def _tc_pre_body(deg_ref, x_ref, w_ref, hs_ref, dis_ref):
    deg = deg_ref[0] + deg_ref[1] + 1.0
    dis = lax.rsqrt(deg)
    dis_ref[...] = dis
    xw = jnp.dot(x_ref[...], w_ref[...], preferred_element_type=jnp.float32)
    hs_ref[...] = xw * dis[:, :1]


def _tc_pre(degpair, x, W0):
    return pl.pallas_call(
        _tc_pre_body,
        grid=(_N // _R,),
        in_specs=[
            pl.BlockSpec((2, _R, 16), lambda i: (0, i, 0)),
            pl.BlockSpec((_R, _D), lambda i: (i, 0)),
            pl.BlockSpec((_D, _D), lambda i: (0, 0)),
        ],
        out_specs=[
            pl.BlockSpec((_R, _D), lambda i: (i, 0)),
            pl.BlockSpec((_R, 16), lambda i: (i, 0)),
        ],
        out_shape=[
            jax.ShapeDtypeStruct((_N, _D), jnp.float32),
            jax.ShapeDtypeStruct((_N, 16), jnp.float32),
        ],
    )(degpair, x, W0)


def _tc_mid_body(p_ref, hs_ref, dis_ref, b_ref, w_ref, o_ref):
    d = dis_ref[:, :1]
    agg = (p_ref[0] + p_ref[1] + hs_ref[...]) * d
    h = jnp.maximum(agg + b_ref[...], 0.0)
    o_ref[...] = jnp.dot(h, w_ref[...], preferred_element_type=jnp.float32) * d


def _tc_mid(p, hs, dis, b, W):
    return pl.pallas_call(
        _tc_mid_body,
        grid=(_N // _R,),
        in_specs=[
            pl.BlockSpec((2, _R, _D), lambda i: (0, i, 0)),
            pl.BlockSpec((_R, _D), lambda i: (i, 0)),
            pl.BlockSpec((_R, 16), lambda i: (i, 0)),
            pl.BlockSpec((1, _D), lambda i: (0, 0)),
            pl.BlockSpec((_D, _D), lambda i: (0, 0)),
        ],
        out_specs=pl.BlockSpec((_R, _D), lambda i: (i, 0)),
        out_shape=jax.ShapeDtypeStruct((_N, _D), jnp.float32),
    )(p, hs, dis, b.reshape(1, _D), W)


def _tc_fin_body(p_ref, hs_ref, dis_ref, b_ref, o_ref):
    d = dis_ref[:, :1]
    z = (p_ref[0] + p_ref[1] + hs_ref[...]) * d + b_ref[...]
    m = jnp.max(z, axis=1, keepdims=True)
    zs = z - m
    o_ref[...] = zs - jnp.log(jnp.sum(jnp.exp(zs), axis=1, keepdims=True))


def _tc_fin(p, hs, dis, b):
    return pl.pallas_call(
        _tc_fin_body,
        grid=(_N // _R,),
        in_specs=[
            pl.BlockSpec((2, _R, _D), lambda i: (0, i, 0)),
            pl.BlockSpec((_R, _D), lambda i: (i, 0)),
            pl.BlockSpec((_R, 16), lambda i: (i, 0)),
            pl.BlockSpec((1, _D), lambda i: (0, 0)),
        ],
        out_specs=pl.BlockSpec((_R, _D), lambda i: (i, 0)),
        out_shape=jax.ShapeDtypeStruct((_N, _D), jnp.float32),
    )(p, hs, dis, b.reshape(1, _D))


def kernel(x, edge_index, W0, b0, W1, b1, W2, b2):
    src = edge_index[0]
    dst = edge_index[1]
    zeros16 = jnp.zeros((_RPT, 16), jnp.float32)
    ones16 = jnp.ones((_CHUNK, 16), jnp.float32)
    zrows = jnp.zeros((_RPT, _D), jnp.float32)

    degpair = _deg_kernel(dst, zeros16, ones16)
    hs0, dis = _tc_pre(degpair, x, W0)
    p0 = _agg_kernel(hs0, src, dst, zrows)
    hs1 = _tc_mid(p0, hs0, dis, b0, W1)
    p1 = _agg_kernel(hs1, src, dst, zrows)
    hs2 = _tc_mid(p1, hs1, dis, b1, W2)
    p2 = _agg_kernel(hs2, src, dst, zrows)
    return _tc_fin(p2, hs2, dis, b2)


# speedup vs baseline: 2.6287x; 1.0759x over previous
import functools

import jax
import jax.numpy as jnp
from jax import lax
from jax.experimental import pallas as pl
from jax.experimental.pallas import tpu as pltpu
from jax.experimental.pallas import tpu_sc as plsc

_N = 10000
_NPAD = 10240
_E = 320000
_D = 128
_CHUNK = 128
_NCHUNKS = _E // _CHUNK
_NW = 32
_RPT = _NPAD // 16

_mesh = plsc.VectorSubcoreMesh(core_axis_name="c", subcore_axis_name="s")


@functools.partial(
    pl.kernel,
    out_type=jax.ShapeDtypeStruct((2, _NPAD, 16), jnp.float32),
    mesh=_mesh,
    compiler_params=pltpu.CompilerParams(use_tc_tiling_on_sc=False),
    scratch_types=[
        pltpu.VMEM_SHARED((_NPAD, 16), jnp.float32),
        pltpu.VMEM((_CHUNK, 16), jnp.float32),
        pltpu.VMEM((1, _CHUNK), jnp.int32),
        pltpu.SemaphoreType.DMA,
    ],
)
def _deg_kernel(dst_hbm, zeros16_hbm, ones16_hbm, out_hbm, acc, ones_v, didx, sem):
    cid = lax.axis_index("c")
    sid = lax.axis_index("s")
    base = sid * _RPT
    pltpu.sync_copy(zeros16_hbm, acc.at[pl.ds(base, _RPT)])
    pltpu.sync_copy(ones16_hbm, ones_v)
    plsc.subcore_barrier()
    wid = sid * 2 + cid

    def body(j, carry):
        off = (wid + j * _NW) * _CHUNK
        pltpu.sync_copy(dst_hbm.at[pl.ds(off, _CHUNK)], didx.at[0])
        pltpu.async_copy(ones_v, acc.at[didx.at[0]], sem, add=True).wait()
        return carry

    lax.fori_loop(0, (_NCHUNKS - wid + _NW - 1) // _NW, body, 0)
    plsc.subcore_barrier()
    pltpu.sync_copy(acc.at[pl.ds(base, _RPT)], out_hbm.at[cid, pl.ds(base, _RPT)])


@functools.partial(
    pl.kernel,
    out_type=jax.ShapeDtypeStruct((2, _NPAD, _D), jnp.float32),
    mesh=_mesh,
    scratch_types=[
        pltpu.VMEM_SHARED((_NPAD, _D), jnp.float32),
        pltpu.VMEM((_CHUNK, _D), jnp.float32),
        pltpu.VMEM((_CHUNK, _D), jnp.float32),
        pltpu.VMEM((1, _CHUNK), jnp.int32),
        pltpu.VMEM((1, _CHUNK), jnp.int32),
        pltpu.VMEM((1, _CHUNK), jnp.int32),
        pltpu.VMEM((1, _CHUNK), jnp.int32),
        pltpu.SemaphoreType.DMA,
        pltpu.SemaphoreType.DMA,
        pltpu.SemaphoreType.DMA,
    ],
)
def _agg_kernel(hs_hbm, src_hbm, dst_hbm, zrows_hbm, out_hbm,
                acc, rows0, rows1, sidx0, didx0, sidx1, didx1,
                gsem0, gsem1, ssem):
    cid = lax.axis_index("c")
    sid = lax.axis_index("s")
    base = sid * _RPT
    pltpu.sync_copy(zrows_hbm, acc.at[pl.ds(base, _RPT)])
    plsc.subcore_barrier()
    wid = sid * 2 + cid

    def stage_idx(j, sidx, didx):
        off = (wid + j * _NW) * _CHUNK
        pltpu.sync_copy(src_hbm.at[pl.ds(off, _CHUNK)], sidx.at[0])
        pltpu.sync_copy(dst_hbm.at[pl.ds(off, _CHUNK)], didx.at[0])

    def body(j2, carry):
        stage_idx(2 * j2, sidx0, didx0)
        stage_idx(2 * j2 + 1, sidx1, didx1)
        ga = pltpu.async_copy(hs_hbm.at[sidx0.at[0]], rows0, gsem0)
        gb = pltpu.async_copy(hs_hbm.at[sidx1.at[0]], rows1, gsem1)
        ga.wait()
        pltpu.async_copy(rows0, acc.at[didx0.at[0]], ssem, add=True).wait()
        gb.wait()
        pltpu.async_copy(rows1, acc.at[didx1.at[0]], ssem, add=True).wait()
        return carry

    lax.fori_loop(0, 39, body, 0)

    @pl.when(wid < _NCHUNKS % _NW)
    def _():
        stage_idx(78, sidx0, didx0)
        pltpu.async_copy(hs_hbm.at[sidx0.at[0]], rows0, gsem0).wait()
        pltpu.async_copy(rows0, acc.at[didx0.at[0]], ssem, add=True).wait()

    plsc.subcore_barrier()
    pltpu.sync_copy(acc.at[pl.ds(base, _RPT)], out_hbm.at[cid, pl.ds(base, _RPT)])


_R = 1000


def _tc_pre_body(deg_ref, x_ref, w_ref, hs_ref, dis_ref):
    deg = deg_ref[0] + deg_ref[1] + 1.0
    dis = lax.rsqrt(deg)
    dis_ref[...] = dis
    xw = jnp.dot(x_ref[...], w_ref[...], preferred_element_type=jnp.float32)
    hs_ref[...] = xw * dis[:, :1]


def _tc_pre(degpair, x, W0):
    return pl.pallas_call(
        _tc_pre_body,
        grid=(_N // _R,),
        in_specs=[
            pl.BlockSpec((2, _R, 16), lambda i: (0, i, 0)),
            pl.BlockSpec((_R, _D), lambda i: (i, 0)),
            pl.BlockSpec((_D, _D), lambda i: (0, 0)),
        ],
        out_specs=[
            pl.BlockSpec((_R, _D), lambda i: (i, 0)),
            pl.BlockSpec((_R, 16), lambda i: (i, 0)),
        ],
        out_shape=[
            jax.ShapeDtypeStruct((_N, _D), jnp.float32),
            jax.ShapeDtypeStruct((_N, 16), jnp.float32),
        ],
    )(degpair, x, W0)


def _tc_mid_body(p_ref, hs_ref, dis_ref, b_ref, w_ref, o_ref):
    d = dis_ref[:, :1]
    agg = (p_ref[0] + p_ref[1] + hs_ref[...]) * d
    h = jnp.maximum(agg + b_ref[...], 0.0)
    o_ref[...] = jnp.dot(h, w_ref[...], preferred_element_type=jnp.float32) * d


def _tc_mid(p, hs, dis, b, W):
    return pl.pallas_call(
        _tc_mid_body,
        grid=(_N // _R,),
        in_specs=[
            pl.BlockSpec((2, _R, _D), lambda i: (0, i, 0)),
            pl.BlockSpec((_R, _D), lambda i: (i, 0)),
            pl.BlockSpec((_R, 16), lambda i: (i, 0)),
            pl.BlockSpec((1, _D), lambda i: (0, 0)),
            pl.BlockSpec((_D, _D), lambda i: (0, 0)),
        ],
        out_specs=pl.BlockSpec((_R, _D), lambda i: (i, 0)),
        out_shape=jax.ShapeDtypeStruct((_N, _D), jnp.float32),
    )(p, hs, dis, b.reshape(1, _D), W)


def _tc_fin_body(p_ref, hs_ref, dis_ref, b_ref, o_ref):
    d = dis_ref[:, :1]
    z = (p_ref[0] + p_ref[1] + hs_ref[...]) * d + b_ref[...]
    m = jnp.max(z, axis=1, keepdims=True)
    zs = z - m
    o_ref[...] = zs - jnp.log(jnp.sum(jnp.exp(zs), axis=1, keepdims=True))


def _tc_fin(p, hs, dis, b):
    return pl.pallas_call(
        _tc_fin_body,
        grid=(_N // _R,),
        in_specs=[
            pl.BlockSpec((2, _R, _D), lambda i: (0, i, 0)),
            pl.BlockSpec((_R, _D), lambda i: (i, 0)),
            pl.BlockSpec((_R, 16), lambda i: (i, 0)),
            pl.BlockSpec((1, _D), lambda i: (0, 0)),
        ],
        out_specs=pl.BlockSpec((_R, _D), lambda i: (i, 0)),
        out_shape=jax.ShapeDtypeStruct((_N, _D), jnp.float32),
    )(p, hs, dis, b.reshape(1, _D))


def kernel(x, edge_index, W0, b0, W1, b1, W2, b2):
    src = edge_index[0]
    dst = edge_index[1]
    zeros16 = jnp.zeros((_RPT, 16), jnp.float32)
    ones16 = jnp.ones((_CHUNK, 16), jnp.float32)
    zrows = jnp.zeros((_RPT, _D), jnp.float32)

    degpair = _deg_kernel(dst, zeros16, ones16)
    hs0, dis = _tc_pre(degpair, x, W0)
    p0 = _agg_kernel(hs0, src, dst, zrows)
    hs1 = _tc_mid(p0, hs0, dis, b0, W1)
    p1 = _agg_kernel(hs1, src, dst, zrows)
    hs2 = _tc_mid(p1, hs1, dis, b1, W2)
    p2 = _agg_kernel(hs2, src, dst, zrows)
    return _tc_fin(p2, hs2, dis, b2)


# speedup vs baseline: 3.0597x; 1.1639x over previous
import functools

import jax
import jax.numpy as jnp
from jax import lax
from jax.experimental import pallas as pl
from jax.experimental.pallas import tpu as pltpu
from jax.experimental.pallas import tpu_sc as plsc

_N = 10000
_NPAD = 10240
_E = 320000
_D = 128
_CHUNK = 128
_NCHUNKS = _E // _CHUNK
_NW = 32
_RPT = _NPAD // 16

_mesh = plsc.VectorSubcoreMesh(core_axis_name="c", subcore_axis_name="s")


@functools.partial(
    pl.kernel,
    out_type=jax.ShapeDtypeStruct((2, _NPAD, 16), jnp.float32),
    mesh=_mesh,
    compiler_params=pltpu.CompilerParams(use_tc_tiling_on_sc=False),
    scratch_types=[
        pltpu.VMEM_SHARED((_NPAD, 16), jnp.float32),
        pltpu.VMEM((_CHUNK, 16), jnp.float32),
        pltpu.VMEM((1, _CHUNK), jnp.int32),
        pltpu.SemaphoreType.DMA,
    ],
)
def _deg_kernel(dst_hbm, zeros16_hbm, ones16_hbm, out_hbm, acc, ones_v, didx, sem):
    cid = lax.axis_index("c")
    sid = lax.axis_index("s")
    base = sid * _RPT
    pltpu.sync_copy(zeros16_hbm, acc.at[pl.ds(base, _RPT)])
    pltpu.sync_copy(ones16_hbm, ones_v)
    plsc.subcore_barrier()
    wid = sid * 2 + cid

    def body(j, carry):
        off = (wid + j * _NW) * _CHUNK
        pltpu.sync_copy(dst_hbm.at[pl.ds(off, _CHUNK)], didx.at[0])
        pltpu.async_copy(ones_v, acc.at[didx.at[0]], sem, add=True).wait()
        return carry

    lax.fori_loop(0, (_NCHUNKS - wid + _NW - 1) // _NW, body, 0)
    plsc.subcore_barrier()
    pltpu.sync_copy(acc.at[pl.ds(base, _RPT)], out_hbm.at[cid, pl.ds(base, _RPT)])


@functools.partial(
    pl.kernel,
    out_type=jax.ShapeDtypeStruct((2, _NPAD, _D), jnp.float32),
    mesh=_mesh,
    scratch_types=[
        pltpu.VMEM_SHARED((_NPAD, _D), jnp.float32),
        pltpu.VMEM((_CHUNK, _D), jnp.float32),
        pltpu.VMEM((_CHUNK, _D), jnp.float32),
        pltpu.VMEM((2, _CHUNK), jnp.int32),
        pltpu.VMEM((2, _CHUNK), jnp.int32),
        pltpu.SemaphoreType.DMA,
        pltpu.SemaphoreType.DMA,
        pltpu.SemaphoreType.DMA,
    ],
)
def _agg_kernel(hs_hbm, sd_hbm, zrows_hbm, out_hbm,
                acc, rows0, rows1, idx0, idx1,
                gsem0, gsem1, ssem):
    cid = lax.axis_index("c")
    sid = lax.axis_index("s")
    base = sid * _RPT
    pltpu.sync_copy(zrows_hbm, acc.at[pl.ds(base, _RPT)])
    plsc.subcore_barrier()
    wid = sid * 2 + cid

    def stage_idx(j, idx):
        pltpu.sync_copy(sd_hbm.at[wid + j * _NW], idx)

    def body(j2, carry):
        stage_idx(2 * j2, idx0)
        stage_idx(2 * j2 + 1, idx1)
        ga = pltpu.async_copy(hs_hbm.at[idx0.at[0]], rows0, gsem0)
        gb = pltpu.async_copy(hs_hbm.at[idx1.at[0]], rows1, gsem1)
        ga.wait()
        pltpu.async_copy(rows0, acc.at[idx0.at[1]], ssem, add=True).wait()
        gb.wait()
        pltpu.async_copy(rows1, acc.at[idx1.at[1]], ssem, add=True).wait()
        return carry

    lax.fori_loop(0, 39, body, 0)

    @pl.when(wid < _NCHUNKS % _NW)
    def _():
        stage_idx(78, idx0)
        pltpu.async_copy(hs_hbm.at[idx0.at[0]], rows0, gsem0).wait()
        pltpu.async_copy(rows0, acc.at[idx0.at[1]], ssem, add=True).wait()

    plsc.subcore_barrier()
    pltpu.sync_copy(acc.at[pl.ds(base, _RPT)], out_hbm.at[cid, pl.ds(base, _RPT)])


_R = 1000


def _tc_pre_body(deg_ref, x_ref, w_ref, hs_ref, dis_ref):
    deg = deg_ref[0] + deg_ref[1] + 1.0
    dis = lax.rsqrt(deg)
    dis_ref[...] = dis
    xw = jnp.dot(x_ref[...], w_ref[...], preferred_element_type=jnp.float32)
    hs_ref[...] = xw * dis[:, :1]


def _tc_pre(degpair, x, W0):
    return pl.pallas_call(
        _tc_pre_body,
        grid=(_N // _R,),
        in_specs=[
            pl.BlockSpec((2, _R, 16), lambda i: (0, i, 0)),
            pl.BlockSpec((_R, _D), lambda i: (i, 0)),
            pl.BlockSpec((_D, _D), lambda i: (0, 0)),
        ],
        out_specs=[
            pl.BlockSpec((_R, _D), lambda i: (i, 0)),
            pl.BlockSpec((_R, 16), lambda i: (i, 0)),
        ],
        out_shape=[
            jax.ShapeDtypeStruct((_N, _D), jnp.float32),
            jax.ShapeDtypeStruct((_N, 16), jnp.float32),
        ],
    )(degpair, x, W0)


def _tc_mid_body(p_ref, hs_ref, dis_ref, b_ref, w_ref, o_ref):
    d = dis_ref[:, :1]
    agg = (p_ref[0] + p_ref[1] + hs_ref[...]) * d
    h = jnp.maximum(agg + b_ref[...], 0.0)
    o_ref[...] = jnp.dot(h, w_ref[...], preferred_element_type=jnp.float32) * d


def _tc_mid(p, hs, dis, b, W):
    return pl.pallas_call(
        _tc_mid_body,
        grid=(_N // _R,),
        in_specs=[
            pl.BlockSpec((2, _R, _D), lambda i: (0, i, 0)),
            pl.BlockSpec((_R, _D), lambda i: (i, 0)),
            pl.BlockSpec((_R, 16), lambda i: (i, 0)),
            pl.BlockSpec((1, _D), lambda i: (0, 0)),
            pl.BlockSpec((_D, _D), lambda i: (0, 0)),
        ],
        out_specs=pl.BlockSpec((_R, _D), lambda i: (i, 0)),
        out_shape=jax.ShapeDtypeStruct((_N, _D), jnp.float32),
    )(p, hs, dis, b.reshape(1, _D), W)


def _tc_fin_body(p_ref, hs_ref, dis_ref, b_ref, o_ref):
    d = dis_ref[:, :1]
    z = (p_ref[0] + p_ref[1] + hs_ref[...]) * d + b_ref[...]
    m = jnp.max(z, axis=1, keepdims=True)
    zs = z - m
    o_ref[...] = zs - jnp.log(jnp.sum(jnp.exp(zs), axis=1, keepdims=True))


def _tc_fin(p, hs, dis, b):
    return pl.pallas_call(
        _tc_fin_body,
        grid=(_N // _R,),
        in_specs=[
            pl.BlockSpec((2, _R, _D), lambda i: (0, i, 0)),
            pl.BlockSpec((_R, _D), lambda i: (i, 0)),
            pl.BlockSpec((_R, 16), lambda i: (i, 0)),
            pl.BlockSpec((1, _D), lambda i: (0, 0)),
        ],
        out_specs=pl.BlockSpec((_R, _D), lambda i: (i, 0)),
        out_shape=jax.ShapeDtypeStruct((_N, _D), jnp.float32),
    )(p, hs, dis, b.reshape(1, _D))


def kernel(x, edge_index, W0, b0, W1, b1, W2, b2):
    src = edge_index[0]
    dst = edge_index[1]
    sd = jnp.stack(
        [src.reshape(_NCHUNKS, _CHUNK), dst.reshape(_NCHUNKS, _CHUNK)], axis=1
    )
    zeros16 = jnp.zeros((_RPT, 16), jnp.float32)
    ones16 = jnp.ones((_CHUNK, 16), jnp.float32)
    zrows = jnp.zeros((_RPT, _D), jnp.float32)

    degpair = _deg_kernel(dst, zeros16, ones16)
    hs0, dis = _tc_pre(degpair, x, W0)
    p0 = _agg_kernel(hs0, sd, zrows)
    hs1 = _tc_mid(p0, hs0, dis, b0, W1)
    p1 = _agg_kernel(hs1, sd, zrows)
    hs2 = _tc_mid(p1, hs1, dis, b1, W2)
    p2 = _agg_kernel(hs2, sd, zrows)
    return _tc_fin(p2, hs2, dis, b2)


# speedup vs baseline: 3.8527x; 1.2592x over previous
import functools

import jax
import jax.numpy as jnp
from jax import lax
from jax.experimental import pallas as pl
from jax.experimental.pallas import tpu as pltpu
from jax.experimental.pallas import tpu_sc as plsc

_N = 10000
_NPAD = 10240
_E = 320000
_D = 128
_CHUNK = 128
_NCHUNKS = _E // _CHUNK
_NW = 32
_RPT = _NPAD // 16

_mesh = plsc.VectorSubcoreMesh(core_axis_name="c", subcore_axis_name="s")


@functools.partial(
    pl.kernel,
    out_type=jax.ShapeDtypeStruct((2, _NPAD, 16), jnp.float32),
    mesh=_mesh,
    compiler_params=pltpu.CompilerParams(use_tc_tiling_on_sc=False),
    scratch_types=[
        pltpu.VMEM_SHARED((_NPAD, 16), jnp.float32),
        pltpu.VMEM((_CHUNK, 16), jnp.float32),
        pltpu.VMEM((1, _CHUNK), jnp.int32),
        pltpu.SemaphoreType.DMA,
    ],
)
def _deg_kernel(dst_hbm, zeros16_hbm, ones16_hbm, out_hbm, acc, ones_v, didx, sem):
    cid = lax.axis_index("c")
    sid = lax.axis_index("s")
    base = sid * _RPT
    pltpu.sync_copy(zeros16_hbm, acc.at[pl.ds(base, _RPT)])
    pltpu.sync_copy(ones16_hbm, ones_v)
    plsc.subcore_barrier()
    wid = sid * 2 + cid

    def body(j, carry):
        off = (wid + j * _NW) * _CHUNK
        pltpu.sync_copy(dst_hbm.at[pl.ds(off, _CHUNK)], didx.at[0])
        pltpu.async_copy(ones_v, acc.at[didx.at[0]], sem, add=True).wait()
        return carry

    lax.fori_loop(0, (_NCHUNKS - wid + _NW - 1) // _NW, body, 0)
    plsc.subcore_barrier()
    pltpu.sync_copy(acc.at[pl.ds(base, _RPT)], out_hbm.at[cid, pl.ds(base, _RPT)])


@functools.partial(
    pl.kernel,
    out_type=jax.ShapeDtypeStruct((2, _NPAD, _D), jnp.float32),
    mesh=_mesh,
    scratch_types=[
        pltpu.VMEM_SHARED((_NPAD, _D), jnp.float32),
        pltpu.VMEM((_CHUNK, _D), jnp.float32),
        pltpu.VMEM((_CHUNK, _D), jnp.float32),
        pltpu.VMEM((2, _CHUNK), jnp.int32),
        pltpu.VMEM((2, _CHUNK), jnp.int32),
        pltpu.VMEM((2, _CHUNK), jnp.int32),
        pltpu.VMEM((2, _CHUNK), jnp.int32),
        pltpu.SemaphoreType.DMA,
        pltpu.SemaphoreType.DMA,
        pltpu.SemaphoreType.DMA,
        pltpu.SemaphoreType.DMA,
        pltpu.SemaphoreType.DMA,
        pltpu.SemaphoreType.DMA,
        pltpu.SemaphoreType.DMA,
    ],
)
def _agg_kernel(hs_hbm, sd_hbm, zrows_hbm, out_hbm,
                acc, rows0, rows1, idx0, idx1, idx2, idx3,
                gsem0, gsem1, ssem, isem0, isem1, isem2, isem3):
    cid = lax.axis_index("c")
    sid = lax.axis_index("s")
    base = sid * _RPT
    pltpu.sync_copy(zrows_hbm, acc.at[pl.ds(base, _RPT)])
    wid = sid * 2 + cid
    idxs = (idx0, idx1, idx2, idx3)
    isems = (isem0, isem1, isem2, isem3)

    def fire_idx(j, t):
        pltpu.async_copy(sd_hbm.at[wid + j * _NW], idxs[t], isems[t])

    def wait_idx(j, t):
        pltpu.make_async_copy(sd_hbm.at[wid + j * _NW], idxs[t],
                              isems[t]).wait()

    def gather(idx, rows, gsem):
        return pltpu.async_copy(hs_hbm.at[idx.at[0]], rows, gsem)

    def scatter(idx, rows):
        pltpu.async_copy(rows, acc.at[idx.at[1]], ssem, add=True).wait()

    for t in range(4):
        fire_idx(t, t)
    plsc.subcore_barrier()

    def body(j2, carry):
        c = j2 * 4
        for t in range(4):
            wait_idx(c + t, t)
        ga = gather(idx0, rows0, gsem0)
        gb = gather(idx1, rows1, gsem1)
        ga.wait()
        scatter(idx0, rows0)
        gc = gather(idx2, rows0, gsem0)
        gb.wait()
        scatter(idx1, rows1)
        gd = gather(idx3, rows1, gsem1)
        gc.wait()
        scatter(idx2, rows0)
        for t in range(3):
            cn = c + 4 + t

            @pl.when(jnp.logical_or(cn < 78, jnp.logical_and(cn < 79, wid < 4)))
            def _():
                fire_idx(cn, t)

        gd.wait()
        scatter(idx3, rows1)

        @pl.when(c + 7 < 78)
        def _():
            fire_idx(c + 7, 3)

        return carry

    lax.fori_loop(0, 19, body, 0)

    wait_idx(76, 0)
    wait_idx(77, 1)
    ga = gather(idx0, rows0, gsem0)
    gb = gather(idx1, rows1, gsem1)
    ga.wait()
    scatter(idx0, rows0)
    gb.wait()
    scatter(idx1, rows1)

    @pl.when(wid < _NCHUNKS % _NW)
    def _():
        wait_idx(78, 2)
        gather(idx2, rows0, gsem0).wait()
        scatter(idx2, rows0)

    plsc.subcore_barrier()
    pltpu.sync_copy(acc.at[pl.ds(base, _RPT)], out_hbm.at[cid, pl.ds(base, _RPT)])


_R = 1000


def _tc_pre_body(deg_ref, x_ref, w_ref, hs_ref, dis_ref):
    deg = deg_ref[0] + deg_ref[1] + 1.0
    dis = lax.rsqrt(deg)
    dis_ref[...] = dis
    xw = jnp.dot(x_ref[...], w_ref[...], preferred_element_type=jnp.float32)
    hs_ref[...] = xw * dis[:, :1]


def _tc_pre(degpair, x, W0):
    return pl.pallas_call(
        _tc_pre_body,
        grid=(_N // _R,),
        in_specs=[
            pl.BlockSpec((2, _R, 16), lambda i: (0, i, 0)),
            pl.BlockSpec((_R, _D), lambda i: (i, 0)),
            pl.BlockSpec((_D, _D), lambda i: (0, 0)),
        ],
        out_specs=[
            pl.BlockSpec((_R, _D), lambda i: (i, 0)),
            pl.BlockSpec((_R, 16), lambda i: (i, 0)),
        ],
        out_shape=[
            jax.ShapeDtypeStruct((_N, _D), jnp.float32),
            jax.ShapeDtypeStruct((_N, 16), jnp.float32),
        ],
    )(degpair, x, W0)


def _tc_mid_body(p_ref, hs_ref, dis_ref, b_ref, w_ref, o_ref):
    d = dis_ref[:, :1]
    agg = (p_ref[0] + p_ref[1] + hs_ref[...]) * d
    h = jnp.maximum(agg + b_ref[...], 0.0)
    o_ref[...] = jnp.dot(h, w_ref[...], preferred_element_type=jnp.float32) * d


def _tc_mid(p, hs, dis, b, W):
    return pl.pallas_call(
        _tc_mid_body,
        grid=(_N // _R,),
        in_specs=[
            pl.BlockSpec((2, _R, _D), lambda i: (0, i, 0)),
            pl.BlockSpec((_R, _D), lambda i: (i, 0)),
            pl.BlockSpec((_R, 16), lambda i: (i, 0)),
            pl.BlockSpec((1, _D), lambda i: (0, 0)),
            pl.BlockSpec((_D, _D), lambda i: (0, 0)),
        ],
        out_specs=pl.BlockSpec((_R, _D), lambda i: (i, 0)),
        out_shape=jax.ShapeDtypeStruct((_N, _D), jnp.float32),
    )(p, hs, dis, b.reshape(1, _D), W)


def _tc_fin_body(p_ref, hs_ref, dis_ref, b_ref, o_ref):
    d = dis_ref[:, :1]
    z = (p_ref[0] + p_ref[1] + hs_ref[...]) * d + b_ref[...]
    m = jnp.max(z, axis=1, keepdims=True)
    zs = z - m
    o_ref[...] = zs - jnp.log(jnp.sum(jnp.exp(zs), axis=1, keepdims=True))


def _tc_fin(p, hs, dis, b):
    return pl.pallas_call(
        _tc_fin_body,
        grid=(_N // _R,),
        in_specs=[
            pl.BlockSpec((2, _R, _D), lambda i: (0, i, 0)),
            pl.BlockSpec((_R, _D), lambda i: (i, 0)),
            pl.BlockSpec((_R, 16), lambda i: (i, 0)),
            pl.BlockSpec((1, _D), lambda i: (0, 0)),
        ],
        out_specs=pl.BlockSpec((_R, _D), lambda i: (i, 0)),
        out_shape=jax.ShapeDtypeStruct((_N, _D), jnp.float32),
    )(p, hs, dis, b.reshape(1, _D))


def kernel(x, edge_index, W0, b0, W1, b1, W2, b2):
    src = edge_index[0]
    dst = edge_index[1]
    sd = jnp.stack(
        [src.reshape(_NCHUNKS, _CHUNK), dst.reshape(_NCHUNKS, _CHUNK)], axis=1
    )
    zeros16 = jnp.zeros((_RPT, 16), jnp.float32)
    ones16 = jnp.ones((_CHUNK, 16), jnp.float32)
    zrows = jnp.zeros((_RPT, _D), jnp.float32)

    degpair = _deg_kernel(dst, zeros16, ones16)
    hs0, dis = _tc_pre(degpair, x, W0)
    p0 = _agg_kernel(hs0, sd, zrows)
    hs1 = _tc_mid(p0, hs0, dis, b0, W1)
    p1 = _agg_kernel(hs1, sd, zrows)
    hs2 = _tc_mid(p1, hs1, dis, b1, W2)
    p2 = _agg_kernel(hs2, sd, zrows)
    return _tc_fin(p2, hs2, dis, b2)
